# Initial kernel scaffold; baseline (speedup 1.0000x reference)
import jax
import jax.numpy as jnp
from jax.experimental import pallas as pl


def kernel(x, adj, W, a, lin_w, lin_b, bn2_gamma, bn2_beta, bn2_mean, bn2_var):
    raise NotImplementedError("write your pallas kernel here")


# speedup vs baseline: 1.7242x; 1.7242x over previous
import functools

import jax
import jax.numpy as jnp
from jax.experimental import pallas as pl

N = 4096
NFEAT = 256
NHID = 64
NHEADS = 4
NCLASS = 128
ALPHA = 0.2

BLOCK_ROWS = 256


def _prep_kernel(x_ref, wc_ref, ac_ref, h_ref, f_ref):
    h = jnp.dot(x_ref[...], wc_ref[...], preferred_element_type=jnp.float32)
    h_ref[...] = h
    f_ref[...] = jnp.dot(h, ac_ref[...], preferred_element_type=jnp.float32)


def _gat_kernel(adj_ref, f_ref, ft_ref, hall_ref, linw_ref, scale_ref,
                bias_ref, o_ref):
    i = pl.program_id(0)
    adj = adj_ref[...]
    acc = jnp.zeros((BLOCK_ROWS, NCLASS), dtype=jnp.float32)
    for hd in range(NHEADS):
        c0 = hd * NHID
        hh = hall_ref[:, c0:c0 + NHID]
        f1 = f_ref[:, hd:hd + 1]
        f2 = ft_ref[NHEADS + hd:NHEADS + hd + 1, :]
        e = f1 + f2
        e = jnp.where(e > 0, e, ALPHA * e)
        e = jnp.where(adj > 0, e, jnp.float32(-9e15))
        m = jnp.max(e, axis=1, keepdims=True)
        p = jnp.exp(e - m)
        s = jnp.sum(p, axis=1, keepdims=True)
        hp = jnp.dot(p, hh, preferred_element_type=jnp.float32) / s
        ehp = jnp.where(hp > 0, hp,
                        jnp.exp(jnp.minimum(hp, 0.0)) - 1.0)
        hblk = hall_ref[pl.ds(i * BLOCK_ROWS, BLOCK_ROWS), c0:c0 + NHID]
        w0 = linw_ref[2 * c0:2 * c0 + NHID, :]
        w1 = linw_ref[2 * c0 + NHID:2 * c0 + 2 * NHID, :]
        acc += jnp.dot(ehp, w0, preferred_element_type=jnp.float32)
        acc += jnp.dot(hblk, w1, preferred_element_type=jnp.float32)
    o_ref[...] = acc * scale_ref[...] + bias_ref[...]


@jax.jit
def kernel(x, adj, W, a, lin_w, lin_b, bn2_gamma, bn2_beta, bn2_mean, bn2_var):
    w_cat = jnp.concatenate([W[hd] for hd in range(NHEADS)], axis=1)
    a_cat = jnp.zeros((NHEADS * NHID, 2 * NHEADS), dtype=jnp.float32)
    for hd in range(NHEADS):
        a_cat = a_cat.at[hd * NHID:(hd + 1) * NHID, hd].set(a[hd, :NHID, 0])
        a_cat = a_cat.at[hd * NHID:(hd + 1) * NHID, NHEADS + hd].set(
            a[hd, NHID:, 0])

    grid_prep = N // 512
    h_all, f = pl.pallas_call(
        _prep_kernel,
        grid=(grid_prep,),
        in_specs=[
            pl.BlockSpec((512, NFEAT), lambda i: (i, 0)),
            pl.BlockSpec((NFEAT, NHEADS * NHID), lambda i: (0, 0)),
            pl.BlockSpec((NHEADS * NHID, 2 * NHEADS), lambda i: (0, 0)),
        ],
        out_specs=[
            pl.BlockSpec((512, NHEADS * NHID), lambda i: (i, 0)),
            pl.BlockSpec((512, 2 * NHEADS), lambda i: (i, 0)),
        ],
        out_shape=[
            jax.ShapeDtypeStruct((N, NHEADS * NHID), jnp.float32),
            jax.ShapeDtypeStruct((N, 2 * NHEADS), jnp.float32),
        ],
    )(x, w_cat, a_cat)

    ft = f.T

    scale = (bn2_gamma / jnp.sqrt(bn2_var + 1e-5)).reshape(1, NCLASS)
    bias = ((lin_b - bn2_mean) * scale[0] + bn2_beta).reshape(1, NCLASS)

    grid = N // BLOCK_ROWS
    y = pl.pallas_call(
        _gat_kernel,
        grid=(grid,),
        in_specs=[
            pl.BlockSpec((BLOCK_ROWS, N), lambda i: (i, 0)),
            pl.BlockSpec((BLOCK_ROWS, 2 * NHEADS), lambda i: (i, 0)),
            pl.BlockSpec((2 * NHEADS, N), lambda i: (0, 0)),
            pl.BlockSpec((N, NHEADS * NHID), lambda i: (0, 0)),
            pl.BlockSpec((2 * NHEADS * NHID, NCLASS), lambda i: (0, 0)),
            pl.BlockSpec((1, NCLASS), lambda i: (0, 0)),
            pl.BlockSpec((1, NCLASS), lambda i: (0, 0)),
        ],
        out_specs=pl.BlockSpec((BLOCK_ROWS, NCLASS), lambda i: (i, 0)),
        out_shape=jax.ShapeDtypeStruct((N, NCLASS), jnp.float32),
    )(adj, f, ft, h_all, lin_w, scale, bias)
    return y


# speedup vs baseline: 2.0483x; 1.1880x over previous
import jax
import jax.numpy as jnp
from jax.experimental import pallas as pl

N = 4096
NFEAT = 256
NHID = 64
NHEADS = 4
NCLASS = 128
ALPHA = 0.2

BLOCK_ROWS = 256
HGRP = 128


def _prep_kernel(x_ref, wc_ref, ac_ref, ones_ref, h_ref, f_ref):
    h = jnp.dot(x_ref[...], wc_ref[...], preferred_element_type=jnp.float32)
    h = h + ones_ref[...]
    h_ref[...] = h
    f = jnp.dot(h, ac_ref[...], preferred_element_type=jnp.float32)
    f_ref[...] = jnp.clip(f, -40.0, 40.0)


def _gat_kernel(adj_ref, f_ref, ft_ref, haug_ref, linw_ref, scale_ref,
                bias_ref, o_ref):
    i = pl.program_id(0)
    adj = adj_ref[...]
    mask = (adj > 0).astype(jnp.float32)
    acc = jnp.zeros((BLOCK_ROWS, NCLASS), dtype=jnp.float32)
    for hd in range(NHEADS):
        g0 = hd * HGRP
        f1 = f_ref[:, hd:hd + 1]
        f2 = ft_ref[NHEADS + hd:NHEADS + hd + 1, :]
        t = f1 + f2
        t = 0.6 * t + 0.4 * jnp.abs(t)
        p = jnp.exp(t) * mask
        hps = jnp.dot(p, haug_ref[:, g0:g0 + HGRP],
                      preferred_element_type=jnp.float32)
        hp = hps[:, :NHID] / hps[:, NHID:NHID + 1]
        ehp = jnp.where(hp > 0, hp,
                        jnp.exp(jnp.minimum(hp, 0.0)) - 1.0)
        hblk = haug_ref[pl.ds(i * BLOCK_ROWS, BLOCK_ROWS), g0:g0 + NHID]
        w0 = linw_ref[2 * hd * NHID:(2 * hd + 1) * NHID, :]
        w1 = linw_ref[(2 * hd + 1) * NHID:(2 * hd + 2) * NHID, :]
        acc += jnp.dot(ehp, w0, preferred_element_type=jnp.float32)
        acc += jnp.dot(hblk, w1, preferred_element_type=jnp.float32)
    o_ref[...] = acc * scale_ref[...] + bias_ref[...]


@jax.jit
def kernel(x, adj, W, a, lin_w, lin_b, bn2_gamma, bn2_beta, bn2_mean, bn2_var):
    w_cat2 = jnp.zeros((NFEAT, NHEADS * HGRP), dtype=jnp.float32)
    ones_row = jnp.zeros((1, NHEADS * HGRP), dtype=jnp.float32)
    a_cat2 = jnp.zeros((NHEADS * HGRP, 2 * NHEADS), dtype=jnp.float32)
    for hd in range(NHEADS):
        g0 = hd * HGRP
        w_cat2 = w_cat2.at[:, g0:g0 + NHID].set(W[hd])
        ones_row = ones_row.at[0, g0 + NHID].set(1.0)
        a_cat2 = a_cat2.at[g0:g0 + NHID, hd].set(a[hd, :NHID, 0])
        a_cat2 = a_cat2.at[g0:g0 + NHID, NHEADS + hd].set(a[hd, NHID:, 0])

    grid_prep = N // 512
    haug, f = pl.pallas_call(
        _prep_kernel,
        grid=(grid_prep,),
        in_specs=[
            pl.BlockSpec((512, NFEAT), lambda i: (i, 0)),
            pl.BlockSpec((NFEAT, NHEADS * HGRP), lambda i: (0, 0)),
            pl.BlockSpec((NHEADS * HGRP, 2 * NHEADS), lambda i: (0, 0)),
            pl.BlockSpec((1, NHEADS * HGRP), lambda i: (0, 0)),
        ],
        out_specs=[
            pl.BlockSpec((512, NHEADS * HGRP), lambda i: (i, 0)),
            pl.BlockSpec((512, 2 * NHEADS), lambda i: (i, 0)),
        ],
        out_shape=[
            jax.ShapeDtypeStruct((N, NHEADS * HGRP), jnp.float32),
            jax.ShapeDtypeStruct((N, 2 * NHEADS), jnp.float32),
        ],
    )(x, w_cat2, a_cat2, ones_row)

    ft = f.T

    scale = (bn2_gamma / jnp.sqrt(bn2_var + 1e-5)).reshape(1, NCLASS)
    bias = ((lin_b - bn2_mean) * scale[0] + bn2_beta).reshape(1, NCLASS)

    grid = N // BLOCK_ROWS
    y = pl.pallas_call(
        _gat_kernel,
        grid=(grid,),
        in_specs=[
            pl.BlockSpec((BLOCK_ROWS, N), lambda i: (i, 0)),
            pl.BlockSpec((BLOCK_ROWS, 2 * NHEADS), lambda i: (i, 0)),
            pl.BlockSpec((2 * NHEADS, N), lambda i: (0, 0)),
            pl.BlockSpec((N, NHEADS * HGRP), lambda i: (0, 0)),
            pl.BlockSpec((2 * NHEADS * NHID, NCLASS), lambda i: (0, 0)),
            pl.BlockSpec((1, NCLASS), lambda i: (0, 0)),
            pl.BlockSpec((1, NCLASS), lambda i: (0, 0)),
        ],
        out_specs=pl.BlockSpec((BLOCK_ROWS, NCLASS), lambda i: (i, 0)),
        out_shape=jax.ShapeDtypeStruct((N, NCLASS), jnp.float32),
    )(adj, f, ft, haug, lin_w, scale, bias)
    return y


# speedup vs baseline: 2.3069x; 1.1262x over previous
import jax
import jax.numpy as jnp
from jax.experimental import pallas as pl

N = 4096
NFEAT = 256
NHID = 64
NHEADS = 4
NCLASS = 128
ALPHA = 0.2

BLOCK_ROWS = 256
HGRP = 128


def _prep_kernel(x_ref, wc_ref, ac_ref, ones_ref, h_ref, f_ref):
    h = jnp.dot(x_ref[...], wc_ref[...], preferred_element_type=jnp.float32)
    h = h + ones_ref[...]
    h_ref[...] = h
    f = jnp.dot(h, ac_ref[...], preferred_element_type=jnp.float32)
    f = jnp.clip(f, -40.0, 40.0)
    f_ref[...] = jnp.concatenate([jnp.exp(f), jnp.exp(0.2 * f)], axis=1)


def _gat_kernel(adj_ref, f_ref, ft_ref, haug_ref, linw_ref, scale_ref,
                bias_ref, o_ref):
    i = pl.program_id(0)
    adj = adj_ref[...]
    mask = (adj > 0).astype(jnp.float32)
    acc = jnp.zeros((BLOCK_ROWS, NCLASS), dtype=jnp.float32)
    for hd in range(NHEADS):
        g0 = hd * HGRP
        e1 = f_ref[:, hd:hd + 1]
        e1s = f_ref[:, 8 + hd:9 + hd]
        e2 = ft_ref[NHEADS + hd:NHEADS + hd + 1, :]
        e2s = ft_ref[12 + hd:13 + hd, :]
        ppos = e1 * e2
        pneg = e1s * e2s
        p = jnp.where(ppos > 1.0, ppos, pneg) * mask
        hps = jnp.dot(p, haug_ref[:, g0:g0 + HGRP],
                      preferred_element_type=jnp.float32)
        hp = hps[:, :NHID] / hps[:, NHID:NHID + 1]
        ehp = jnp.where(hp > 0, hp,
                        jnp.exp(jnp.minimum(hp, 0.0)) - 1.0)
        hblk = haug_ref[pl.ds(i * BLOCK_ROWS, BLOCK_ROWS), g0:g0 + NHID]
        w0 = linw_ref[2 * hd * NHID:(2 * hd + 1) * NHID, :]
        w1 = linw_ref[(2 * hd + 1) * NHID:(2 * hd + 2) * NHID, :]
        acc += jnp.dot(ehp, w0, preferred_element_type=jnp.float32)
        acc += jnp.dot(hblk, w1, preferred_element_type=jnp.float32)
    o_ref[...] = acc * scale_ref[...] + bias_ref[...]


@jax.jit
def kernel(x, adj, W, a, lin_w, lin_b, bn2_gamma, bn2_beta, bn2_mean, bn2_var):
    w_cat2 = jnp.zeros((NFEAT, NHEADS * HGRP), dtype=jnp.float32)
    ones_row = jnp.zeros((1, NHEADS * HGRP), dtype=jnp.float32)
    a_cat2 = jnp.zeros((NHEADS * HGRP, 2 * NHEADS), dtype=jnp.float32)
    for hd in range(NHEADS):
        g0 = hd * HGRP
        w_cat2 = w_cat2.at[:, g0:g0 + NHID].set(W[hd])
        ones_row = ones_row.at[0, g0 + NHID].set(1.0)
        a_cat2 = a_cat2.at[g0:g0 + NHID, hd].set(a[hd, :NHID, 0])
        a_cat2 = a_cat2.at[g0:g0 + NHID, NHEADS + hd].set(a[hd, NHID:, 0])

    grid_prep = N // 512
    haug, f = pl.pallas_call(
        _prep_kernel,
        grid=(grid_prep,),
        in_specs=[
            pl.BlockSpec((512, NFEAT), lambda i: (i, 0)),
            pl.BlockSpec((NFEAT, NHEADS * HGRP), lambda i: (0, 0)),
            pl.BlockSpec((NHEADS * HGRP, 2 * NHEADS), lambda i: (0, 0)),
            pl.BlockSpec((1, NHEADS * HGRP), lambda i: (0, 0)),
        ],
        out_specs=[
            pl.BlockSpec((512, NHEADS * HGRP), lambda i: (i, 0)),
            pl.BlockSpec((512, 4 * NHEADS), lambda i: (i, 0)),
        ],
        out_shape=[
            jax.ShapeDtypeStruct((N, NHEADS * HGRP), jnp.float32),
            jax.ShapeDtypeStruct((N, 4 * NHEADS), jnp.float32),
        ],
    )(x, w_cat2, a_cat2, ones_row)

    ft = f.T

    scale = (bn2_gamma / jnp.sqrt(bn2_var + 1e-5)).reshape(1, NCLASS)
    bias = ((lin_b - bn2_mean) * scale[0] + bn2_beta).reshape(1, NCLASS)

    grid = N // BLOCK_ROWS
    y = pl.pallas_call(
        _gat_kernel,
        grid=(grid,),
        in_specs=[
            pl.BlockSpec((BLOCK_ROWS, N), lambda i: (i, 0)),
            pl.BlockSpec((BLOCK_ROWS, 4 * NHEADS), lambda i: (i, 0)),
            pl.BlockSpec((4 * NHEADS, N), lambda i: (0, 0)),
            pl.BlockSpec((N, NHEADS * HGRP), lambda i: (0, 0)),
            pl.BlockSpec((2 * NHEADS * NHID, NCLASS), lambda i: (0, 0)),
            pl.BlockSpec((1, NCLASS), lambda i: (0, 0)),
            pl.BlockSpec((1, NCLASS), lambda i: (0, 0)),
        ],
        out_specs=pl.BlockSpec((BLOCK_ROWS, NCLASS), lambda i: (i, 0)),
        out_shape=jax.ShapeDtypeStruct((N, NCLASS), jnp.float32),
    )(adj, f, ft, haug, lin_w, scale, bias)
    return y


# speedup vs baseline: 2.3608x; 1.0234x over previous
import jax
import jax.numpy as jnp
from jax.experimental import pallas as pl

N = 4096
NFEAT = 256
NHID = 64
NHEADS = 4
NCLASS = 128
ALPHA = 0.2

BLOCK_ROWS = 256
HGRP = 128


def _prep_kernel(x_ref, wc_ref, ac_ref, ones_ref, h_ref, f_ref):
    h = jnp.dot(x_ref[...], wc_ref[...], preferred_element_type=jnp.float32)
    h = h + ones_ref[...]
    h_ref[...] = h
    f = jnp.dot(h, ac_ref[...], preferred_element_type=jnp.float32)
    f = jnp.clip(f, -40.0, 40.0)
    f_ref[...] = jnp.concatenate([jnp.exp(f), jnp.exp(0.2 * f)], axis=1)


def _gat_kernel(adj_ref, f_ref, ft_ref, haug_ref, linw_ref, scale_ref,
                bias_ref, o_ref):
    i = pl.program_id(0)
    adj = adj_ref[...]
    mask = (adj > 0).astype(jnp.float32)
    acc = jnp.zeros((BLOCK_ROWS, NCLASS), dtype=jnp.float32)
    for hd in range(NHEADS):
        g0 = hd * HGRP
        e1 = f_ref[:, hd:hd + 1]
        e1s = f_ref[:, 8 + hd:9 + hd]
        e2 = ft_ref[NHEADS + hd:NHEADS + hd + 1, :]
        e2s = ft_ref[12 + hd:13 + hd, :]
        p = jnp.maximum(e1 * e2, e1s * e2s) * mask
        hps = jnp.dot(p, haug_ref[:, g0:g0 + HGRP],
                      preferred_element_type=jnp.float32)
        hp = hps[:, :NHID] / hps[:, NHID:NHID + 1]
        ehp = jnp.where(hp > 0, hp,
                        jnp.exp(jnp.minimum(hp, 0.0)) - 1.0)
        hblk = haug_ref[pl.ds(i * BLOCK_ROWS, BLOCK_ROWS), g0:g0 + NHID]
        w0 = linw_ref[2 * hd * NHID:(2 * hd + 1) * NHID, :]
        w1 = linw_ref[(2 * hd + 1) * NHID:(2 * hd + 2) * NHID, :]
        acc += jnp.dot(ehp, w0, preferred_element_type=jnp.float32)
        acc += jnp.dot(hblk, w1, preferred_element_type=jnp.float32)
    o_ref[...] = acc * scale_ref[...] + bias_ref[...]


@jax.jit
def kernel(x, adj, W, a, lin_w, lin_b, bn2_gamma, bn2_beta, bn2_mean, bn2_var):
    w_cat2 = jnp.zeros((NFEAT, NHEADS * HGRP), dtype=jnp.float32)
    ones_row = jnp.zeros((1, NHEADS * HGRP), dtype=jnp.float32)
    a_cat2 = jnp.zeros((NHEADS * HGRP, 2 * NHEADS), dtype=jnp.float32)
    for hd in range(NHEADS):
        g0 = hd * HGRP
        w_cat2 = w_cat2.at[:, g0:g0 + NHID].set(W[hd])
        ones_row = ones_row.at[0, g0 + NHID].set(1.0)
        a_cat2 = a_cat2.at[g0:g0 + NHID, hd].set(a[hd, :NHID, 0])
        a_cat2 = a_cat2.at[g0:g0 + NHID, NHEADS + hd].set(a[hd, NHID:, 0])

    grid_prep = N // 512
    haug, f = pl.pallas_call(
        _prep_kernel,
        grid=(grid_prep,),
        in_specs=[
            pl.BlockSpec((512, NFEAT), lambda i: (i, 0)),
            pl.BlockSpec((NFEAT, NHEADS * HGRP), lambda i: (0, 0)),
            pl.BlockSpec((NHEADS * HGRP, 2 * NHEADS), lambda i: (0, 0)),
            pl.BlockSpec((1, NHEADS * HGRP), lambda i: (0, 0)),
        ],
        out_specs=[
            pl.BlockSpec((512, NHEADS * HGRP), lambda i: (i, 0)),
            pl.BlockSpec((512, 4 * NHEADS), lambda i: (i, 0)),
        ],
        out_shape=[
            jax.ShapeDtypeStruct((N, NHEADS * HGRP), jnp.float32),
            jax.ShapeDtypeStruct((N, 4 * NHEADS), jnp.float32),
        ],
    )(x, w_cat2, a_cat2, ones_row)

    ft = f.T

    scale = (bn2_gamma / jnp.sqrt(bn2_var + 1e-5)).reshape(1, NCLASS)
    bias = ((lin_b - bn2_mean) * scale[0] + bn2_beta).reshape(1, NCLASS)

    grid = N // BLOCK_ROWS
    y = pl.pallas_call(
        _gat_kernel,
        grid=(grid,),
        in_specs=[
            pl.BlockSpec((BLOCK_ROWS, N), lambda i: (i, 0)),
            pl.BlockSpec((BLOCK_ROWS, 4 * NHEADS), lambda i: (i, 0)),
            pl.BlockSpec((4 * NHEADS, N), lambda i: (0, 0)),
            pl.BlockSpec((N, NHEADS * HGRP), lambda i: (0, 0)),
            pl.BlockSpec((2 * NHEADS * NHID, NCLASS), lambda i: (0, 0)),
            pl.BlockSpec((1, NCLASS), lambda i: (0, 0)),
            pl.BlockSpec((1, NCLASS), lambda i: (0, 0)),
        ],
        out_specs=pl.BlockSpec((BLOCK_ROWS, NCLASS), lambda i: (i, 0)),
        out_shape=jax.ShapeDtypeStruct((N, NCLASS), jnp.float32),
    )(adj, f, ft, haug, lin_w, scale, bias)
    return y


# speedup vs baseline: 3.3035x; 1.3993x over previous
import jax
import jax.numpy as jnp
from jax import lax
from jax.experimental import pallas as pl

N = 4096
NFEAT = 256
NHID = 64
NHEADS = 4
NCLASS = 128
ALPHA = 0.2

BLOCK_ROWS = 256
BX = 512
HGRP = 128

_DN_NT = (((1,), (1,)), ((), ()))


def _prep_kernel(x_ref, w_ref, a_ref, h_ref, f_ref, ft_ref):
    x = x_ref[...]
    for hd in range(NHEADS):
        g0 = hd * HGRP
        h = jnp.dot(x, w_ref[hd], preferred_element_type=jnp.float32)
        h_ref[:, g0:g0 + NHID] = h
        h_ref[:, g0 + NHID:g0 + NHID + 1] = jnp.ones((BX, 1), jnp.float32)
        h_ref[:, g0 + NHID + 1:g0 + HGRP] = jnp.zeros(
            (BX, HGRP - NHID - 1), jnp.float32)
        a1 = a_ref[hd:hd + 1, :NHID]
        a2 = a_ref[hd:hd + 1, NHID:]
        f1 = lax.dot_general(h, a1, _DN_NT,
                             preferred_element_type=jnp.float32)
        f1 = jnp.clip(f1, -40.0, 40.0)
        f_ref[:, hd:hd + 1] = jnp.exp(f1)
        f_ref[:, NHEADS + hd:NHEADS + hd + 1] = jnp.exp(0.2 * f1)
        f2 = lax.dot_general(a2, h, _DN_NT,
                             preferred_element_type=jnp.float32)
        f2 = jnp.clip(f2, -40.0, 40.0)
        ft_ref[hd:hd + 1, :] = jnp.exp(f2)
        ft_ref[NHEADS + hd:NHEADS + hd + 1, :] = jnp.exp(0.2 * f2)


def _gat_kernel(adj_ref, f_ref, ft_ref, haug_ref, linw_ref, linb_ref,
                gam_ref, bet_ref, mu_ref, var_ref, o_ref):
    i = pl.program_id(0)
    adj = adj_ref[...]
    mask = (adj > 0).astype(jnp.float32)
    acc = jnp.zeros((BLOCK_ROWS, NCLASS), dtype=jnp.float32)
    for hd in range(NHEADS):
        g0 = hd * HGRP
        e1 = f_ref[:, hd:hd + 1]
        e1s = f_ref[:, NHEADS + hd:NHEADS + hd + 1]
        e2 = ft_ref[hd:hd + 1, :]
        e2s = ft_ref[NHEADS + hd:NHEADS + hd + 1, :]
        p = jnp.maximum(e1 * e2, e1s * e2s) * mask
        hps = jnp.dot(p, haug_ref[:, g0:g0 + HGRP],
                      preferred_element_type=jnp.float32)
        hp = hps[:, :NHID] / hps[:, NHID:NHID + 1]
        ehp = jnp.where(hp > 0, hp,
                        jnp.exp(jnp.minimum(hp, 0.0)) - 1.0)
        hblk = haug_ref[pl.ds(i * BLOCK_ROWS, BLOCK_ROWS), g0:g0 + NHID]
        w0 = linw_ref[2 * hd * NHID:(2 * hd + 1) * NHID, :]
        w1 = linw_ref[(2 * hd + 1) * NHID:(2 * hd + 2) * NHID, :]
        acc += jnp.dot(ehp, w0, preferred_element_type=jnp.float32)
        acc += jnp.dot(hblk, w1, preferred_element_type=jnp.float32)
    scale = gam_ref[...] * lax.rsqrt(var_ref[...] + 1e-5)
    bias = (linb_ref[...] - mu_ref[...]) * scale + bet_ref[...]
    o_ref[...] = acc * scale + bias


@jax.jit
def kernel(x, adj, W, a, lin_w, lin_b, bn2_gamma, bn2_beta, bn2_mean, bn2_var):
    a2 = a.reshape(NHEADS, 2 * NHID)

    grid_prep = N // BX
    haug, f, ft = pl.pallas_call(
        _prep_kernel,
        grid=(grid_prep,),
        in_specs=[
            pl.BlockSpec((BX, NFEAT), lambda i: (i, 0)),
            pl.BlockSpec((NHEADS, NFEAT, NHID), lambda i: (0, 0, 0)),
            pl.BlockSpec((NHEADS, 2 * NHID), lambda i: (0, 0)),
        ],
        out_specs=[
            pl.BlockSpec((BX, NHEADS * HGRP), lambda i: (i, 0)),
            pl.BlockSpec((BX, 2 * NHEADS), lambda i: (i, 0)),
            pl.BlockSpec((2 * NHEADS, BX), lambda i: (0, i)),
        ],
        out_shape=[
            jax.ShapeDtypeStruct((N, NHEADS * HGRP), jnp.float32),
            jax.ShapeDtypeStruct((N, 2 * NHEADS), jnp.float32),
            jax.ShapeDtypeStruct((2 * NHEADS, N), jnp.float32),
        ],
    )(x, W, a2)

    row = lambda v: v.reshape(1, NCLASS)

    grid = N // BLOCK_ROWS
    y = pl.pallas_call(
        _gat_kernel,
        grid=(grid,),
        in_specs=[
            pl.BlockSpec((BLOCK_ROWS, N), lambda i: (i, 0)),
            pl.BlockSpec((BLOCK_ROWS, 2 * NHEADS), lambda i: (i, 0)),
            pl.BlockSpec((2 * NHEADS, N), lambda i: (0, 0)),
            pl.BlockSpec((N, NHEADS * HGRP), lambda i: (0, 0)),
            pl.BlockSpec((2 * NHEADS * NHID, NCLASS), lambda i: (0, 0)),
        ] + [pl.BlockSpec((1, NCLASS), lambda i: (0, 0))] * 5,
        out_specs=pl.BlockSpec((BLOCK_ROWS, NCLASS), lambda i: (i, 0)),
        out_shape=jax.ShapeDtypeStruct((N, NCLASS), jnp.float32),
    )(adj, f, ft, haug, lin_w, row(lin_b), row(bn2_gamma), row(bn2_beta),
      row(bn2_mean), row(bn2_var))
    return y


# speedup vs baseline: 3.3831x; 1.0241x over previous
import jax
import jax.numpy as jnp
from jax import lax
from jax.experimental import pallas as pl

N = 4096
NFEAT = 256
NHID = 64
NHEADS = 4
NCLASS = 128
ALPHA = 0.2

BLOCK_ROWS = 256
BX = 512
HGRP = 128

_DN_NT = (((1,), (1,)), ((), ()))


def _prep_kernel(x_ref, w_ref, a_ref, hb_ref, hc_ref, f_ref, ft_ref):
    x = x_ref[...]
    for hd in range(NHEADS):
        g0 = hd * HGRP
        h = jnp.dot(x, w_ref[hd], preferred_element_type=jnp.float32)
        hc_ref[:, hd * NHID:(hd + 1) * NHID] = h
        hb_ref[:, g0:g0 + NHID] = h.astype(jnp.bfloat16)
        hb_ref[:, g0 + NHID:g0 + NHID + 1] = jnp.ones((BX, 1), jnp.bfloat16)
        hb_ref[:, g0 + NHID + 1:g0 + HGRP] = jnp.zeros(
            (BX, HGRP - NHID - 1), jnp.bfloat16)
        a1 = a_ref[hd:hd + 1, :NHID]
        a2 = a_ref[hd:hd + 1, NHID:]
        f1 = lax.dot_general(h, a1, _DN_NT,
                             preferred_element_type=jnp.float32)
        f1 = jnp.clip(f1, -40.0, 40.0)
        f_ref[:, hd:hd + 1] = jnp.exp(f1).astype(jnp.bfloat16)
        f_ref[:, NHEADS + hd:NHEADS + hd + 1] = jnp.exp(
            0.2 * f1).astype(jnp.bfloat16)
        f2 = lax.dot_general(a2, h, _DN_NT,
                             preferred_element_type=jnp.float32)
        f2 = jnp.clip(f2, -40.0, 40.0)
        ft_ref[hd:hd + 1, :] = jnp.exp(f2).astype(jnp.bfloat16)
        ft_ref[NHEADS + hd:NHEADS + hd + 1, :] = jnp.exp(
            0.2 * f2).astype(jnp.bfloat16)


def _gat_kernel(adj_ref, f_ref, ft_ref, haug_ref, hc_ref, linw_ref, linb_ref,
                gam_ref, bet_ref, mu_ref, var_ref, o_ref):
    adj = adj_ref[...]
    mask = (adj > 0).astype(jnp.bfloat16)
    acc = jnp.zeros((BLOCK_ROWS, NCLASS), dtype=jnp.float32)
    for hd in range(NHEADS):
        g0 = hd * HGRP
        e1 = f_ref[:, hd:hd + 1]
        e1s = f_ref[:, NHEADS + hd:NHEADS + hd + 1]
        e2 = ft_ref[hd:hd + 1, :]
        e2s = ft_ref[NHEADS + hd:NHEADS + hd + 1, :]
        p = jnp.maximum(e1 * e2, e1s * e2s) * mask
        hps = jnp.dot(p, haug_ref[:, g0:g0 + HGRP],
                      preferred_element_type=jnp.float32)
        hp = hps[:, :NHID] / hps[:, NHID:NHID + 1]
        ehp = jnp.where(hp > 0, hp,
                        jnp.exp(jnp.minimum(hp, 0.0)) - 1.0)
        hblk = hc_ref[:, hd * NHID:(hd + 1) * NHID]
        w0 = linw_ref[2 * hd * NHID:(2 * hd + 1) * NHID, :]
        w1 = linw_ref[(2 * hd + 1) * NHID:(2 * hd + 2) * NHID, :]
        acc += jnp.dot(ehp, w0, preferred_element_type=jnp.float32)
        acc += jnp.dot(hblk, w1, preferred_element_type=jnp.float32)
    scale = gam_ref[...] * lax.rsqrt(var_ref[...] + 1e-5)
    bias = (linb_ref[...] - mu_ref[...]) * scale + bet_ref[...]
    o_ref[...] = acc * scale + bias


@jax.jit
def kernel(x, adj, W, a, lin_w, lin_b, bn2_gamma, bn2_beta, bn2_mean, bn2_var):
    a2 = a.reshape(NHEADS, 2 * NHID)

    grid_prep = N // BX
    haug, hcat, f, ft = pl.pallas_call(
        _prep_kernel,
        grid=(grid_prep,),
        in_specs=[
            pl.BlockSpec((BX, NFEAT), lambda i: (i, 0)),
            pl.BlockSpec((NHEADS, NFEAT, NHID), lambda i: (0, 0, 0)),
            pl.BlockSpec((NHEADS, 2 * NHID), lambda i: (0, 0)),
        ],
        out_specs=[
            pl.BlockSpec((BX, NHEADS * HGRP), lambda i: (i, 0)),
            pl.BlockSpec((BX, NHEADS * NHID), lambda i: (i, 0)),
            pl.BlockSpec((BX, 2 * NHEADS), lambda i: (i, 0)),
            pl.BlockSpec((2 * NHEADS, BX), lambda i: (0, i)),
        ],
        out_shape=[
            jax.ShapeDtypeStruct((N, NHEADS * HGRP), jnp.bfloat16),
            jax.ShapeDtypeStruct((N, NHEADS * NHID), jnp.float32),
            jax.ShapeDtypeStruct((N, 2 * NHEADS), jnp.bfloat16),
            jax.ShapeDtypeStruct((2 * NHEADS, N), jnp.bfloat16),
        ],
    )(x, W, a2)

    row = lambda v: v.reshape(1, NCLASS)

    grid = N // BLOCK_ROWS
    y = pl.pallas_call(
        _gat_kernel,
        grid=(grid,),
        in_specs=[
            pl.BlockSpec((BLOCK_ROWS, N), lambda i: (i, 0)),
            pl.BlockSpec((BLOCK_ROWS, 2 * NHEADS), lambda i: (i, 0)),
            pl.BlockSpec((2 * NHEADS, N), lambda i: (0, 0)),
            pl.BlockSpec((N, NHEADS * HGRP), lambda i: (0, 0)),
            pl.BlockSpec((BLOCK_ROWS, NHEADS * NHID), lambda i: (i, 0)),
            pl.BlockSpec((2 * NHEADS * NHID, NCLASS), lambda i: (0, 0)),
        ] + [pl.BlockSpec((1, NCLASS), lambda i: (0, 0))] * 5,
        out_specs=pl.BlockSpec((BLOCK_ROWS, NCLASS), lambda i: (i, 0)),
        out_shape=jax.ShapeDtypeStruct((N, NCLASS), jnp.float32),
    )(adj, f, ft, haug, hcat, lin_w, row(lin_b), row(bn2_gamma),
      row(bn2_beta), row(bn2_mean), row(bn2_var))
    return y
